# Initial kernel scaffold; baseline (speedup 1.0000x reference)
import jax
import jax.numpy as jnp
from jax.experimental import pallas as pl


def kernel(codes, indices, memory_bank, centroids):
    raise NotImplementedError("write your pallas kernel here")


# speedup vs baseline: 1.0001x; 1.0001x over previous
import jax
import jax.numpy as jnp
from jax.experimental import pallas as pl

TEMPERATURE = 0.07
KNNS = 50


def _l2norm(x, eps=1e-12):
    return x / jnp.maximum(jnp.linalg.norm(x, axis=-1, keepdims=True), eps)


def kernel(codes, indices, memory_bank, centroids):
    v = _l2norm(codes)
    code_data = jax.lax.stop_gradient(v)
    mem = memory_bank.at[indices].set(code_data)
    mem = jax.lax.stop_gradient(_l2norm(mem))
    n_batch = codes.shape[0]
    n_bank = mem.shape[0]
    sims_nn = code_data @ mem.T
    _, nn_idx = jax.lax.top_k(sims_nn, KNNS + 1)
    bg_neighbours = jnp.zeros((n_batch, n_bank), dtype=bool).at[jnp.arange(n_batch)[:, None], nn_idx].set(True)
    c = _l2norm(centroids)
    labels = jnp.argmax(mem @ c.T, axis=1)
    close_neighbours = labels[indices][:, None] == labels[None, :]
    neighbour_intersect = jnp.logical_and(bg_neighbours, close_neighbours)
    exp_vals = jnp.exp((v @ mem.T) / TEMPERATURE)
    d1 = jnp.sum(exp_vals * bg_neighbours.astype(exp_vals.dtype), axis=1)
    d2 = jnp.sum(exp_vals * neighbour_intersect.astype(exp_vals.dtype), axis=1)
    loss = jnp.sum(jnp.log(d1) - jnp.log(d2)) / n_batch
    return loss


# speedup vs baseline: 3.6681x; 3.6678x over previous
import functools

import jax
import jax.numpy as jnp
from jax.experimental import pallas as pl
from jax.experimental.pallas import tpu as pltpu

_T = 0.07
_K = 51
_B = 1024
_N = 100000
_D = 128
_NC = 100
_TILE = 1024
_NBT = 98
_NT = _NBT + 1
_CW = 32
_NCHUNK = _NT * _TILE // _CW
_PADCHUNK = _N // _CW
_NEG = -1e30


def _rownorm(x):
    n = jnp.sqrt(jnp.sum(x * x, axis=-1, keepdims=True))
    return x / jnp.maximum(n, 1e-12)


def _labels_of(rows_b16, cent_b16):
    sc = jax.lax.dot_general(rows_b16, cent_b16,
                             (((1,), (1,)), ((), ())),
                             preferred_element_type=jnp.float32)
    lane = jax.lax.broadcasted_iota(jnp.int32, sc.shape, 1).astype(jnp.float32)
    sc = jnp.where(lane >= _NC, _NEG, sc)
    m = jnp.max(sc, axis=1, keepdims=True)
    lab = jnp.min(jnp.where(sc == m, lane, 1e9), axis=1, keepdims=True)
    return lab


def _k1_body(codes_ref, bank_ref, cent_ref, idxc_ref, idxr_ref,
             s_ref, m_ref, lab_ref, labq_ref):
    t = pl.program_id(0)
    codes = codes_ref[...]
    v = _rownorm(codes)
    vb = v.astype(jnp.bfloat16)
    cent = _rownorm(cent_ref[...])
    cb = cent.astype(jnp.bfloat16)
    idxc = idxc_ref[...]
    lane_i = jax.lax.broadcasted_iota(jnp.int32, (1, _TILE), 1)
    subl_i = jax.lax.broadcasted_iota(jnp.int32, (_B, 1), 0)

    def emit(y, colmask):
        yb = y.astype(jnp.bfloat16)
        lab_ref[0, :, :] = _labels_of(yb, cb)
        s = jax.lax.dot_general(vb, yb, (((1,), (1,)), ((), ())),
                                preferred_element_type=jnp.float32)
        s = jnp.where(colmask, _NEG, s)
        s_ref[...] = s
        parts = [jnp.max(s[:, k * _CW:(k + 1) * _CW], axis=1, keepdims=True)
                 for k in range(_TILE // _CW)]
        m_ref[0] = jnp.concatenate(parts, axis=1)

    @pl.when(t < _NBT)
    def _bank():
        x = bank_ref[0]
        y = _rownorm(x)
        colids = lane_i + t * _TILE
        owned = jnp.max(jnp.where(idxc == colids, 1.0, 0.0),
                        axis=0, keepdims=True) > 0.0
        emit(y, jnp.logical_or(owned, colids >= _N))

    @pl.when(t == _NBT)
    def _virtual():
        vn2 = _rownorm(v)
        idxr = idxr_ref[0]
        eqq = idxc == idxr
        loser = jnp.max(jnp.where(jnp.logical_and(eqq, subl_i > lane_i),
                                  1.0, 0.0), axis=0, keepdims=True) > 0.0
        emit(vn2, loser)
        w = jnp.max(jnp.where(eqq, lane_i, -1), axis=1, keepdims=True)
        lv = _labels_of(vn2.astype(jnp.bfloat16), cb)
        oh = (lane_i == w).astype(jnp.float32)
        labq_ref[...] = jax.lax.dot_general(
            oh, lv, (((1,), (0,)), ((), ())),
            preferred_element_type=jnp.float32)


def _k2_body(m_ref, ids_ref, ms_ref):
    ms_ref[...] = m_ref[...]
    lane = jax.lax.broadcasted_iota(jnp.int32, (_B, _NCHUNK), 1).astype(jnp.float32)
    for k in range(_K):
        cur = ms_ref[...]
        m = jnp.max(cur, axis=1, keepdims=True)
        gid = jnp.min(jnp.where(cur == m, lane, 4e9), axis=1, keepdims=True)
        ids_ref[:, k] = gid[:, 0]
        ms_ref[...] = jnp.where(lane == gid, _NEG, cur)
    for k in range(_K, 64):
        ids_ref[:, k] = jnp.full((_B,), float(_PADCHUNK), jnp.float32)


def _k3a_body(s_ref, lab_ref, ids_ref, pool_ref, pooll_ref, poolg_ref):
    for k in range(64):
        cidf = ids_ref[0, 0, k]
        cid = cidf.astype(jnp.int32)
        pool_ref[0, k, :] = s_ref[0, cid, :]
        pooll_ref[0, k, :] = lab_ref[0, cid, :]
        poolg_ref[0, k, :] = jnp.full((_CW,), cidf * float(_CW), jnp.float32)


def _k3b_body(pool_ref, pooll_ref, poolg_ref, labq_ref, out_ref, *, rblk):
    i = pl.program_id(0)
    x = pool_ref[...]
    lane = jax.lax.broadcasted_iota(jnp.int32, (rblk, 64, _CW), 2).astype(jnp.float32)
    g = poolg_ref[...] + lane
    lq = labq_ref[...].reshape(rblk, 1, 1)
    lmatch = pooll_ref[...] == lq
    d1 = jnp.zeros((rblk, 1, 1), jnp.float32)
    d2 = jnp.zeros((rblk, 1, 1), jnp.float32)
    for _ in range(_K):
        m = jnp.max(x, axis=(1, 2), keepdims=True)
        e = jnp.exp(m / _T)
        eq = x == m
        gm = jnp.min(jnp.where(eq, g, 4e9), axis=(1, 2), keepdims=True)
        sel = jnp.logical_and(eq, g == gm)
        hit = jnp.max(jnp.where(jnp.logical_and(sel, lmatch), 1.0, 0.0),
                      axis=(1, 2), keepdims=True) > 0.0
        d1 = d1 + e
        d2 = d2 + jnp.where(hit, e, 0.0)
        x = jnp.where(sel, _NEG, x)
    r = jnp.log(d1) - jnp.log(d2)
    part = jnp.sum(r) / float(_B)

    @pl.when(i == 0)
    def _init():
        out_ref[...] = jnp.zeros((1, 1), jnp.float32)

    out_ref[...] += jnp.reshape(part, (1, 1))


@jax.jit
def kernel(codes, indices, memory_bank, centroids):
    f32 = jnp.float32
    bank3 = jnp.pad(memory_bank, ((0, _NBT * _TILE - _N), (0, 0))
                    ).reshape(_NBT, _TILE, _D)
    cent_p = jnp.pad(centroids, ((0, 128 - _NC), (0, 0)))
    idxc = indices.reshape(_B, 1)
    idxr3 = indices.reshape(1, 1, _B)

    s, m, lab, labq = pl.pallas_call(
        _k1_body,
        grid=(_NT,),
        in_specs=[
            pl.BlockSpec((_B, _D), lambda t: (0, 0)),
            pl.BlockSpec((1, _TILE, _D), lambda t: (jnp.minimum(t, _NBT - 1), 0, 0)),
            pl.BlockSpec((128, _D), lambda t: (0, 0)),
            pl.BlockSpec((_B, 1), lambda t: (0, 0)),
            pl.BlockSpec((1, 1, _B), lambda t: (0, 0, 0)),
        ],
        out_specs=[
            pl.BlockSpec((_B, _TILE), lambda t: (0, t)),
            pl.BlockSpec((1, _B, _TILE // _CW), lambda t: (t, 0, 0)),
            pl.BlockSpec((1, _TILE, 1), lambda t: (t, 0, 0)),
            pl.BlockSpec((_B, 1), lambda t: (0, 0)),
        ],
        out_shape=[
            jax.ShapeDtypeStruct((_B, _NT * _TILE), f32),
            jax.ShapeDtypeStruct((_NT, _B, _TILE // _CW), f32),
            jax.ShapeDtypeStruct((_NT, _TILE, 1), f32),
            jax.ShapeDtypeStruct((_B, 1), f32),
        ],
    )(codes, bank3, cent_p, idxc, idxr3)

    ids = pl.pallas_call(
        _k2_body,
        in_specs=[pl.BlockSpec((_B, _NCHUNK), lambda: (0, 0))],
        out_specs=pl.BlockSpec((_B, 64), lambda: (0, 0)),
        out_shape=jax.ShapeDtypeStruct((_B, 64), f32),
        scratch_shapes=[pltpu.VMEM((_B, _NCHUNK), f32)],
    )(jnp.swapaxes(m, 0, 1).reshape(_B, _NCHUNK))

    pool, pooll, poolg = pl.pallas_call(
        _k3a_body,
        grid=(_B,),
        in_specs=[
            pl.BlockSpec((1, _NCHUNK, _CW), lambda i: (i, 0, 0)),
            pl.BlockSpec((1, _NCHUNK, _CW), lambda i: (0, 0, 0)),
            pl.BlockSpec((1, 1, 64), lambda i: (i, 0, 0)),
        ],
        out_specs=[
            pl.BlockSpec((1, 64, _CW), lambda i: (i, 0, 0)),
            pl.BlockSpec((1, 64, _CW), lambda i: (i, 0, 0)),
            pl.BlockSpec((1, 64, _CW), lambda i: (i, 0, 0)),
        ],
        out_shape=[
            jax.ShapeDtypeStruct((_B, 64, _CW), f32),
            jax.ShapeDtypeStruct((_B, 64, _CW), f32),
            jax.ShapeDtypeStruct((_B, 64, _CW), f32),
        ],
    )(s.reshape(_B, _NCHUNK, _CW), lab.reshape(1, _NCHUNK, _CW),
      ids.reshape(_B, 1, 64))

    rblk = 8
    loss = pl.pallas_call(
        functools.partial(_k3b_body, rblk=rblk),
        grid=(_B // rblk,),
        in_specs=[
            pl.BlockSpec((rblk, 64, _CW), lambda i: (i, 0, 0)),
            pl.BlockSpec((rblk, 64, _CW), lambda i: (i, 0, 0)),
            pl.BlockSpec((rblk, 64, _CW), lambda i: (i, 0, 0)),
            pl.BlockSpec((rblk, 1), lambda i: (i, 0)),
        ],
        out_specs=pl.BlockSpec((1, 1), lambda i: (0, 0)),
        out_shape=jax.ShapeDtypeStruct((1, 1), f32),
    )(pool, pooll, poolg, labq)

    return loss.reshape(())


# speedup vs baseline: 4.1108x; 1.1207x over previous
import functools

import jax
import jax.numpy as jnp
from jax.experimental import pallas as pl
from jax.experimental.pallas import tpu as pltpu

_T = 0.07
_K = 51
_B = 1024
_N = 100000
_D = 128
_NC = 100
_TILE = 1024
_NBT = 98
_NT = _NBT + 1
_CW = 32
_NCHUNK = _NT * _TILE // _CW
_PADCHUNK = _N // _CW
_NEG = -1e30
_POOL = 56


def _rownorm(x):
    n = jnp.sqrt(jnp.sum(x * x, axis=-1, keepdims=True))
    return x / jnp.maximum(n, 1e-12)


def _labels_of(rows_b16, cent_b16):
    sc = jax.lax.dot_general(rows_b16, cent_b16,
                             (((1,), (1,)), ((), ())),
                             preferred_element_type=jnp.float32)
    lane = jax.lax.broadcasted_iota(jnp.int32, sc.shape, 1).astype(jnp.float32)
    sc = jnp.where(lane >= _NC, _NEG, sc)
    m = jnp.max(sc, axis=1, keepdims=True)
    lab = jnp.min(jnp.where(sc == m, lane, 1e9), axis=1, keepdims=True)
    return lab


def _k1_body(codes_ref, bank_ref, cent_ref, idxc_ref, idxr_ref,
             s_ref, m_ref, lab_ref, labq_ref):
    t = pl.program_id(0)
    codes = codes_ref[...]
    v = _rownorm(codes)
    vb = v.astype(jnp.bfloat16)
    cent = _rownorm(cent_ref[...])
    cb = cent.astype(jnp.bfloat16)
    idxc = idxc_ref[...]
    lane_i = jax.lax.broadcasted_iota(jnp.int32, (1, _TILE), 1)
    subl_i = jax.lax.broadcasted_iota(jnp.int32, (_B, 1), 0)

    def emit(y, colmask):
        yb = y.astype(jnp.bfloat16)
        lab_ref[0, :, :] = _labels_of(yb, cb)
        s = jax.lax.dot_general(vb, yb, (((1,), (1,)), ((), ())),
                                preferred_element_type=jnp.float32)
        s = jnp.where(colmask, _NEG, s)
        s_ref[...] = s
        parts = [jnp.max(s[:, k * _CW:(k + 1) * _CW], axis=1, keepdims=True)
                 for k in range(_TILE // _CW)]
        m_ref[0] = jnp.concatenate(parts, axis=1)

    @pl.when(t < _NBT)
    def _bank():
        x = bank_ref[0]
        y = _rownorm(x)
        colids = lane_i + t * _TILE
        owned = jnp.max(jnp.where(idxc == colids, 1.0, 0.0),
                        axis=0, keepdims=True) > 0.0
        emit(y, jnp.logical_or(owned, colids >= _N))

    @pl.when(t == _NBT)
    def _virtual():
        vn2 = _rownorm(v)
        idxr = idxr_ref[0]
        eqq = idxc == idxr
        loser = jnp.max(jnp.where(jnp.logical_and(eqq, subl_i > lane_i),
                                  1.0, 0.0), axis=0, keepdims=True) > 0.0
        emit(vn2, loser)
        w = jnp.max(jnp.where(eqq, lane_i, -1), axis=1, keepdims=True)
        lv = _labels_of(vn2.astype(jnp.bfloat16), cb)
        oh = (lane_i == w).astype(jnp.float32)
        labq_ref[...] = jax.lax.dot_general(
            oh, lv, (((1,), (0,)), ((), ())),
            preferred_element_type=jnp.float32)


def _k2_body(m_ref, ids_ref, ms_ref):
    ms_ref[...] = m_ref[...]
    lane = jax.lax.broadcasted_iota(jnp.int32, (_B, _NCHUNK), 1).astype(jnp.float32)
    for k in range(_K):
        cur = ms_ref[...]
        m = jnp.max(cur, axis=1, keepdims=True)
        gid = jnp.min(jnp.where(cur == m, lane, 4e9), axis=1, keepdims=True)
        ids_ref[:, k] = gid[:, 0]
        ms_ref[...] = jnp.where(lane == gid, _NEG, cur)
    for k in range(_K, _POOL):
        ids_ref[:, k] = jnp.full((_B,), float(_PADCHUNK), jnp.float32)


def _k3a_body(s_ref, lab_ref, ids_ref, pool_ref, pooll_ref, poolg_ref):
    for r in range(8):
        for k in range(_POOL):
            cidf = ids_ref[0, r, k]
            cid = cidf.astype(jnp.int32)
            pool_ref[r, k, :] = s_ref[r, cid, :]
            pooll_ref[r, k, :] = lab_ref[0, cid, :]
            poolg_ref[r, k, :] = jnp.full((_CW,), cidf * float(_CW), jnp.float32)


def _k3b_body(pool_ref, pooll_ref, poolg_ref, labq_ref, out_ref, *, rblk):
    i = pl.program_id(0)
    x = pool_ref[...]
    lane = jax.lax.broadcasted_iota(jnp.int32, (rblk, _POOL, _CW), 2).astype(jnp.float32)
    g = poolg_ref[...] + lane
    lq = labq_ref[...].reshape(rblk, 1, 1)
    lmatch = pooll_ref[...] == lq
    d1 = jnp.zeros((rblk, 1, 1), jnp.float32)
    d2 = jnp.zeros((rblk, 1, 1), jnp.float32)
    for _ in range(_K):
        m = jnp.max(x, axis=(1, 2), keepdims=True)
        e = jnp.exp(m / _T)
        eq = x == m
        gm = jnp.min(jnp.where(eq, g, 4e9), axis=(1, 2), keepdims=True)
        sel = jnp.logical_and(eq, g == gm)
        hit = jnp.max(jnp.where(jnp.logical_and(sel, lmatch), 1.0, 0.0),
                      axis=(1, 2), keepdims=True) > 0.0
        d1 = d1 + e
        d2 = d2 + jnp.where(hit, e, 0.0)
        x = jnp.where(sel, _NEG, x)
    r = jnp.log(d1) - jnp.log(d2)
    part = jnp.sum(r) / float(_B)

    @pl.when(i == 0)
    def _init():
        out_ref[...] = jnp.zeros((1, 1), jnp.float32)

    out_ref[...] += jnp.reshape(part, (1, 1))


@jax.jit
def kernel(codes, indices, memory_bank, centroids):
    f32 = jnp.float32
    bank3 = jnp.pad(memory_bank, ((0, _NBT * _TILE - _N), (0, 0))
                    ).reshape(_NBT, _TILE, _D)
    cent_p = jnp.pad(centroids, ((0, 128 - _NC), (0, 0)))
    idxc = indices.reshape(_B, 1)
    idxr3 = indices.reshape(1, 1, _B)

    s, m, lab, labq = pl.pallas_call(
        _k1_body,
        grid=(_NT,),
        in_specs=[
            pl.BlockSpec((_B, _D), lambda t: (0, 0)),
            pl.BlockSpec((1, _TILE, _D), lambda t: (jnp.minimum(t, _NBT - 1), 0, 0)),
            pl.BlockSpec((128, _D), lambda t: (0, 0)),
            pl.BlockSpec((_B, 1), lambda t: (0, 0)),
            pl.BlockSpec((1, 1, _B), lambda t: (0, 0, 0)),
        ],
        out_specs=[
            pl.BlockSpec((_B, _TILE), lambda t: (0, t)),
            pl.BlockSpec((1, _B, _TILE // _CW), lambda t: (t, 0, 0)),
            pl.BlockSpec((1, _TILE, 1), lambda t: (t, 0, 0)),
            pl.BlockSpec((_B, 1), lambda t: (0, 0)),
        ],
        out_shape=[
            jax.ShapeDtypeStruct((_B, _NT * _TILE), f32),
            jax.ShapeDtypeStruct((_NT, _B, _TILE // _CW), f32),
            jax.ShapeDtypeStruct((_NT, _TILE, 1), f32),
            jax.ShapeDtypeStruct((_B, 1), f32),
        ],
    )(codes, bank3, cent_p, idxc, idxr3)

    ids = pl.pallas_call(
        _k2_body,
        in_specs=[pl.BlockSpec((_B, _NCHUNK), lambda: (0, 0))],
        out_specs=pl.BlockSpec((_B, _POOL), lambda: (0, 0)),
        out_shape=jax.ShapeDtypeStruct((_B, _POOL), f32),
        scratch_shapes=[pltpu.VMEM((_B, _NCHUNK), f32)],
    )(jnp.swapaxes(m, 0, 1).reshape(_B, _NCHUNK))

    pool, pooll, poolg = pl.pallas_call(
        _k3a_body,
        grid=(_B // 8,),
        in_specs=[
            pl.BlockSpec((8, _NCHUNK, _CW), lambda i: (i, 0, 0)),
            pl.BlockSpec((1, _NCHUNK, _CW), lambda i: (0, 0, 0)),
            pl.BlockSpec((1, 8, _POOL), lambda i: (i, 0, 0)),
        ],
        out_specs=[
            pl.BlockSpec((8, _POOL, _CW), lambda i: (i, 0, 0)),
            pl.BlockSpec((8, _POOL, _CW), lambda i: (i, 0, 0)),
            pl.BlockSpec((8, _POOL, _CW), lambda i: (i, 0, 0)),
        ],
        out_shape=[
            jax.ShapeDtypeStruct((_B, _POOL, _CW), f32),
            jax.ShapeDtypeStruct((_B, _POOL, _CW), f32),
            jax.ShapeDtypeStruct((_B, _POOL, _CW), f32),
        ],
    )(s.reshape(_B, _NCHUNK, _CW), lab.reshape(1, _NCHUNK, _CW),
      ids.reshape(_B // 8, 8, _POOL))

    rblk = 8
    loss = pl.pallas_call(
        functools.partial(_k3b_body, rblk=rblk),
        grid=(_B // rblk,),
        in_specs=[
            pl.BlockSpec((rblk, _POOL, _CW), lambda i: (i, 0, 0)),
            pl.BlockSpec((rblk, _POOL, _CW), lambda i: (i, 0, 0)),
            pl.BlockSpec((rblk, _POOL, _CW), lambda i: (i, 0, 0)),
            pl.BlockSpec((rblk, 1), lambda i: (i, 0)),
        ],
        out_specs=pl.BlockSpec((1, 1), lambda i: (0, 0)),
        out_shape=jax.ShapeDtypeStruct((1, 1), f32),
    )(pool, pooll, poolg, labq)

    return loss.reshape(())
